# baseline (speedup 1.0000x reference)
import functools

import jax
import jax.numpy as jnp
from jax import lax
from jax.experimental import pallas as pl
from jax.experimental.pallas import tpu as pltpu
from jax.experimental.pallas import tpu_sc as plsc

L = 16
CHUNK = 128
NBUF = 3
BL = 125


@functools.lru_cache(maxsize=None)
def _build_transpose(V, F, nc, ns):
    mesh = plsc.VectorSubcoreMesh(core_axis_name="c", subcore_axis_name="s")
    W = 2 * F
    PAN = 128
    nw = nc * ns
    p_full = V // PAN
    tail = V - p_full * PAN
    base = p_full // nw
    extra = p_full % nw
    fv = F // L

    @functools.partial(
        pl.kernel,
        mesh=mesh,
        compiler_params=pltpu.CompilerParams(
            needs_layout_passes=False, use_tc_tiling_on_sc=True),
        out_type=jax.ShapeDtypeStruct((V * F,), jnp.float32),
        scratch_types=[
            pltpu.VMEM((F, PAN), jnp.float32),
            pltpu.VMEM((PAN // 2 * W,), jnp.float32),
        ],
    )
    def k(tT, t64, out, ib, ob):
        wid = lax.axis_index("s") * nc + lax.axis_index("c")
        pstart = base * wid + jnp.minimum(wid, extra)
        npan = base + (wid < extra).astype(jnp.int32)
        iota = lax.iota(jnp.int32, L)

        def transpose_panel(n_items):
            def row(r, c2):
                for v in range(W // L):
                    fvec = iota + (v % fv) * L
                    ivec = jnp.broadcast_to(
                        (2 * r + v // fv).astype(jnp.int32), (L,))
                    vals = plsc.load_gather(ib, [fvec, ivec])
                    ob[pl.ds(r * W + v * L, L)] = vals
                return c2
            lax.fori_loop(0, n_items // 2, row, 0)

        def panel(p, carry):
            pid = pstart + p

            @pl.when(p < npan)
            def _():
                col = pl.multiple_of(pid * PAN, PAN)
                pltpu.sync_copy(tT.at[:, pl.ds(col, PAN)], ib)
                transpose_panel(PAN)
                pltpu.sync_copy(
                    ob, out.at[pl.ds(pid * (PAN // 2) * W, PAN // 2 * W)])
            return carry

        lax.fori_loop(0, base + (1 if extra else 0), panel, 0)

        if tail:
            @pl.when(wid == nw - 1)
            def _():
                pltpu.sync_copy(
                    t64, out.at[pl.ds(p_full * (PAN // 2) * W, tail * F)])

    return k


@functools.lru_cache(maxsize=None)
def _build_gather(total, per_tile, n_chunks, F, nc, ns):
    mesh = plsc.VectorSubcoreMesh(core_axis_name="c", subcore_axis_name="s")
    pv = CHUNK // L
    W = 2 * F

    @functools.partial(
        pl.kernel,
        mesh=mesh,
        compiler_params=pltpu.CompilerParams(
            needs_layout_passes=False, use_tc_tiling_on_sc=True),
        out_type=jax.ShapeDtypeStruct((total,), jnp.float32),
        scratch_types=[
            pltpu.VMEM((n_chunks, CHUNK), jnp.int32),
            pltpu.VMEM((n_chunks, CHUNK), jnp.int32),
            pltpu.VMEM((per_tile,), jnp.int32),
            pltpu.VMEM((per_tile,), jnp.int32),
            pltpu.VMEM((per_tile,), jnp.int32),
            pltpu.VMEM((NBUF, CHUNK, W), jnp.float32),
            pltpu.VMEM((NBUF, CHUNK, W), jnp.float32),
            pltpu.VMEM((2 * F,), jnp.float32),
            pltpu.VMEM((per_tile,), jnp.float32),
        ] + [pltpu.SemaphoreType.DMA] * (2 * NBUF),
    )
    def k(table, rtab, hrow, trow, hoff_hbm, toff_hbm, rel_hbm, out,
          hrows, trows, hoff, toff, relv, hbuf, tbuf, rbuf, outbuf, *sems):
        hsem = sems[:NBUF]
        tsem = sems[NBUF:]
        wid = lax.axis_index("s") * nc + lax.axis_index("c")
        base = wid * per_tile
        crow = wid * n_chunks
        pltpu.sync_copy(rtab, rbuf)
        pltpu.sync_copy(hrow.at[pl.ds(crow, n_chunks)], hrows)
        pltpu.sync_copy(trow.at[pl.ds(crow, n_chunks)], trows)
        pltpu.sync_copy(hoff_hbm.at[pl.ds(base, per_tile)], hoff)
        pltpu.sync_copy(toff_hbm.at[pl.ds(base, per_tile)], toff)
        pltpu.sync_copy(rel_hbm.at[pl.ds(base, per_tile)], relv)
        iota = lax.iota(jnp.int32, L)
        rows = [iota + p * L for p in range(pv)]

        def fire(c):
            b = c % NBUF
            pltpu.async_copy(table.at[hrows.at[c]], hbuf.at[b], hsem[b])
            pltpu.async_copy(table.at[trows.at[c]], tbuf.at[b], tsem[b])

        def drain(c):
            b = c % NBUF
            pltpu.make_async_copy(
                table.at[hrows.at[c]], hbuf.at[b], hsem[b]).wait()
            pltpu.make_async_copy(
                table.at[trows.at[c]], tbuf.at[b], tsem[b]).wait()

        for c in range(NBUF - 1):
            fire(c)
        for c in range(n_chunks):
            if c + NBUF - 1 < n_chunks:
                fire(c + NBUF - 1)
            drain(c)
            hb = hbuf.at[c % NBUF]
            tb = tbuf.at[c % NBUF]
            sl = lambda p: pl.ds(c * CHUNK + p * L, L)
            rels = [relv[sl(p)] * F for p in range(pv)]
            hoffs = [hoff[sl(p)] for p in range(pv)]
            toffs = [toff[sl(p)] for p in range(pv)]

            def body(f, accs):
                fs = jnp.broadcast_to(f.astype(jnp.int32), (L,))
                new = []
                for p in range(pv):
                    hv = plsc.load_gather(hb, [rows[p], hoffs[p] + fs])
                    tv = plsc.load_gather(tb, [rows[p], toffs[p] + fs])
                    rv = plsc.load_gather(rbuf, [rels[p] + fs])
                    d = hv + rv - tv
                    new.append(accs[p] + d * d)
                return tuple(new)

            accs = lax.fori_loop(
                0, F, body,
                tuple(jnp.zeros((L,), jnp.float32) for _ in range(pv)))
            for p in range(pv):
                outbuf[pl.ds(c * CHUNK + p * L, L)] = -accs[p]
        pltpu.sync_copy(outbuf, out.at[pl.ds(base, per_tile)])

    return k


def kernel(itemEmbedding, r_table, head_ids, tail_ids, relation_ids):
    B, K = head_ids.shape
    total = B * K
    V, F = itemEmbedding.shape
    info = plsc.get_sparse_core_info()
    nc, ns = info.num_cores, info.num_subcores
    per_tile = total // (nc * ns)
    n_chunks = per_tile // CHUNK
    p_full = V // 128
    t64 = lax.slice(itemEmbedding, (p_full * 128, 0), (V, F)).reshape(-1)
    tableL = _build_transpose(V, F, nc, ns)(
        jnp.transpose(itemEmbedding), t64).reshape(V // 2, 2 * F)
    hids = head_ids.reshape(-1).astype(jnp.int32)
    tids = tail_ids.reshape(-1).astype(jnp.int32)
    out = _build_gather(total, per_tile, n_chunks, F, nc, ns)(
        tableL, r_table.reshape(-1),
        (hids >> 1).reshape(total // CHUNK, CHUNK),
        (tids >> 1).reshape(total // CHUNK, CHUNK),
        (hids & 1) * F,
        (tids & 1) * F,
        relation_ids.reshape(-1).astype(jnp.int32))
    return out.reshape(B, K)


# speedup vs baseline: 3.2467x; 3.2467x over previous
import functools

import jax
import jax.numpy as jnp
from jax import lax
from jax.experimental import pallas as pl
from jax.experimental.pallas import tpu as pltpu
from jax.experimental.pallas import tpu_sc as plsc

L = 16
CHUNK = 128


@functools.lru_cache(maxsize=None)
def _build(total, per_tile, n_chunks, F, nc, ns):
    mesh = plsc.VectorSubcoreMesh(core_axis_name="c", subcore_axis_name="s")
    pv = CHUNK // L

    @functools.partial(
        pl.kernel,
        mesh=mesh,
        compiler_params=pltpu.CompilerParams(
            needs_layout_passes=False, use_tc_tiling_on_sc=True),
        out_type=jax.ShapeDtypeStruct((total,), jnp.float32),
        scratch_types=[
            pltpu.VMEM((per_tile,), jnp.int32),
            pltpu.VMEM((per_tile,), jnp.int32),
            pltpu.VMEM((per_tile,), jnp.int32),
            pltpu.VMEM((CHUNK, F), jnp.float32),
            pltpu.VMEM((CHUNK, F), jnp.float32),
            pltpu.VMEM((CHUNK, F), jnp.float32),
            pltpu.VMEM((CHUNK, F), jnp.float32),
            pltpu.VMEM((CHUNK, F), jnp.float32),
            pltpu.VMEM((CHUNK, F), jnp.float32),
            pltpu.VMEM((2 * F,), jnp.float32),
            pltpu.VMEM((per_tile,), jnp.float32),
            pltpu.SemaphoreType.DMA,
            pltpu.SemaphoreType.DMA,
            pltpu.SemaphoreType.DMA,
            pltpu.SemaphoreType.DMA,
            pltpu.SemaphoreType.DMA,
            pltpu.SemaphoreType.DMA,
        ],
    )
    def k(table, rtab, hids, tids, rids, out,
          relv, hidv, tidv, hb0, hb1, hb2, tb0, tb1, tb2,
          rbuf, outbuf, hs0, hs1, hs2, ts0, ts1, ts2):
        wid = lax.axis_index("s") * nc + lax.axis_index("c")
        base = wid * per_tile
        pltpu.sync_copy(rtab, rbuf)
        pltpu.sync_copy(rids.at[pl.ds(base, per_tile)], relv)
        pltpu.sync_copy(hids.at[pl.ds(base, per_tile)], hidv)
        pltpu.sync_copy(tids.at[pl.ds(base, per_tile)], tidv)
        iota = lax.iota(jnp.int32, L)
        rows = [iota + p * L for p in range(pv)]

        hbufs = [hb0, hb1, hb2]
        tbufs = [tb0, tb1, tb2]
        hsems = [hs0, hs1, hs2]
        tsems = [ts0, ts1, ts2]

        def fire(c):
            b = c % 3
            off = c * CHUNK
            hb, tb, hsem, tsem = hbufs[b], tbufs[b], hsems[b], tsems[b]

            def issue(g, carry):
                j0 = g * L
                hvec = hidv[pl.ds(off + j0, L)]
                tvec = tidv[pl.ds(off + j0, L)]
                hrows_s = [
                    lax.reduce_sum(jnp.where(iota == lane, hvec, 0), axes=(0,))
                    for lane in range(L)]
                trows_s = [
                    lax.reduce_sum(jnp.where(iota == lane, tvec, 0), axes=(0,))
                    for lane in range(L)]
                for lane in range(L):
                    pltpu.async_copy(
                        table.at[pl.ds(hrows_s[lane], 1)],
                        hb.at[pl.ds(j0 + lane, 1)], hsem)
                    pltpu.async_copy(
                        table.at[pl.ds(trows_s[lane], 1)],
                        tb.at[pl.ds(j0 + lane, 1)], tsem)
                return carry

            lax.fori_loop(0, CHUNK // L, issue, 0)

        def drain(c):
            b = c % 3
            pltpu.make_async_copy(
                table.at[pl.ds(0, CHUNK)], hbufs[b], hsems[b]).wait()
            pltpu.make_async_copy(
                table.at[pl.ds(0, CHUNK)], tbufs[b], tsems[b]).wait()

        def compute(c, hb, tb):
            rels = [relv[pl.ds(c * CHUNK + p * L, L)] * F for p in range(pv)]

            def body(f, accs):
                fs = jnp.broadcast_to(f.astype(jnp.int32), (L,))
                new = []
                for p in range(pv):
                    hv = plsc.load_gather(hb, [rows[p], fs])
                    tv = plsc.load_gather(tb, [rows[p], fs])
                    rv = plsc.load_gather(rbuf, [rels[p] + fs])
                    d = hv + rv - tv
                    new.append(accs[p] + d * d)
                return tuple(new)

            accs = lax.fori_loop(
                0, F, body,
                tuple(jnp.zeros((L,), jnp.float32) for _ in range(pv)))
            for p in range(pv):
                outbuf[pl.ds(c * CHUNK + p * L, L)] = -accs[p]

        fire(0)
        fire(1)
        for c in range(n_chunks):
            if c + 2 < n_chunks:
                fire(c + 2)
            drain(c)
            compute(c, hbufs[c % 3], tbufs[c % 3])
        pltpu.sync_copy(outbuf, out.at[pl.ds(base, per_tile)])

    return k


def kernel(itemEmbedding, r_table, head_ids, tail_ids, relation_ids):
    B, K = head_ids.shape
    total = B * K
    F = itemEmbedding.shape[1]
    info = plsc.get_sparse_core_info()
    nc, ns = info.num_cores, info.num_subcores
    per_tile = total // (nc * ns)
    n_chunks = per_tile // CHUNK
    k = _build(total, per_tile, n_chunks, F, nc, ns)
    out = k(itemEmbedding, r_table.reshape(-1),
            head_ids.reshape(-1).astype(jnp.int32),
            tail_ids.reshape(-1).astype(jnp.int32),
            relation_ids.reshape(-1).astype(jnp.int32))
    return out.reshape(B, K)
